# Initial kernel scaffold; baseline (speedup 1.0000x reference)
import jax
import jax.numpy as jnp
from jax.experimental import pallas as pl


def kernel(length, emb_table):
    raise NotImplementedError("write your pallas kernel here")


# speedup vs baseline: 9.8163x; 9.8163x over previous
import functools

import jax
import jax.numpy as jnp
from jax import lax
from jax.experimental import pallas as pl
from jax.experimental.pallas import tpu as pltpu
from jax.experimental.pallas import tpu_sc as plsc

MAX_REL = 32
D = 16
L = 2048
TABLE_ROWS = 2 * MAX_REL + 1
STRIP_ROWS = 2 * L - 1
HEAD_ROWS = L - 1 - MAX_REL
TAIL_BASE_ROW = HEAD_ROWS + TABLE_ROWS
STRIP_PAD_ROWS = 4096

NUM_CORES = 2
NUM_SUBCORES = 16
NUM_WORKERS = NUM_CORES * NUM_SUBCORES
ROWS_PER_WORKER = L // NUM_WORKERS
WAVE = 16

_mesh = plsc.VectorSubcoreMesh(
    core_axis_name="c",
    subcore_axis_name="s",
    num_cores=NUM_CORES,
    num_subcores=NUM_SUBCORES,
)


@functools.partial(
    pl.kernel,
    out_type=jax.ShapeDtypeStruct((L * L * D,), jnp.float32),
    mesh=_mesh,
    scratch_types=[
        pltpu.VMEM((TABLE_ROWS * D,), jnp.float32),
        pltpu.VMEM((STRIP_PAD_ROWS * D,), jnp.float32),
        pltpu.SemaphoreType.DMA,
    ],
)
def _rpe_sc(table_hbm, out_hbm, table_v, strip_v, sem):
    wid = lax.axis_index("s") * NUM_CORES + lax.axis_index("c")

    pltpu.sync_copy(table_hbm, table_v)
    row0 = table_v[pl.ds(0, D)]
    row_last = table_v[pl.ds((TABLE_ROWS - 1) * D, D)]

    def fill(i, carry):
        base = i * 8 * D
        for u in range(8):
            off = base + u * D
            strip_v[pl.ds(off, D)] = row0
            strip_v[pl.ds(TAIL_BASE_ROW * D + off, D)] = row_last
        return carry

    lax.fori_loop(0, 252, fill, 0)

    for t in range(TABLE_ROWS):
        strip_v[pl.ds((HEAD_ROWS + t) * D, D)] = table_v[pl.ds(t * D, D)]

    base_row = wid * ROWS_PER_WORKER
    for g in range(0, ROWS_PER_WORKER, WAVE):
        copies = []
        for r in range(WAVE):
            row = base_row + g + r
            src_off = (L - 1 - row) * D
            copies.append(
                pltpu.async_copy(
                    strip_v.at[pl.ds(src_off, L * D)],
                    out_hbm.at[pl.ds(row * (L * D), L * D)],
                    sem,
                )
            )
        for cp in copies:
            cp.wait()


def kernel(length, emb_table):
    del length
    out = _rpe_sc(emb_table.reshape(TABLE_ROWS * D))
    return out.reshape(L, L, D)


# speedup vs baseline: 95.9468x; 9.7742x over previous
import functools

import jax
import jax.numpy as jnp
from jax import lax
from jax.experimental import pallas as pl
from jax.experimental.pallas import tpu as pltpu
from jax.experimental.pallas import tpu_sc as plsc

MAX_REL = 32
D = 16
L = 2048
TABLE_ROWS = 2 * MAX_REL + 1
CHUNK = 16
NUM_CORES = 2
NUM_SUBCORES = 16
INFLIGHT = 16

_mesh = plsc.VectorSubcoreMesh(
    core_axis_name="c",
    subcore_axis_name="s",
    num_cores=NUM_CORES,
    num_subcores=NUM_SUBCORES,
)


@functools.partial(
    pl.kernel,
    out_type=jax.ShapeDtypeStruct((L, D, L), jnp.float32),
    mesh=_mesh,
    compiler_params=pltpu.CompilerParams(needs_layout_passes=False),
    scratch_types=[
        pltpu.VMEM((TABLE_ROWS * D,), jnp.float32),
        pltpu.VMEM((CHUNK, 8, 128), jnp.float32),
        pltpu.VMEM((CHUNK, 8, 128), jnp.float32),
        pltpu.VMEM((CHUNK, 8, 128), jnp.float32),
        pltpu.SemaphoreType.DMA,
        pltpu.SemaphoreType.DMA,
    ],
)
def _rpe_sc(table_hbm, out_hbm, table_v, clo_v, chi_v, band_v, sem_c, sem_b):
    t = lax.axis_index("s")
    s = lax.axis_index("c")
    ds0 = pl.multiple_of(8 * s, 8)
    dj0 = pl.multiple_of(128 * t, 128)

    pltpu.sync_copy(table_hbm, table_v)

    tb_lo = jnp.maximum(128 * t - MAX_REL, 0)
    tb_hi = jnp.minimum(128 * t + 128 + MAX_REL, L)
    n_lo = tb_lo // CHUNK
    n_hi = (L - tb_hi) // CHUNK
    n_band = (tb_hi - tb_lo) // CHUNK

    def build_const(rr, carry):
        for dr in range(8):
            lane = jnp.full((D,), 8 * s + dr, jnp.int32)
            v_lo = plsc.load_gather(table_v, [lane])
            v_hi = plsc.load_gather(table_v, [(TABLE_ROWS - 1) * D + lane])
            for u in range(8):
                clo_v[rr, dr, pl.ds(16 * u, 16)] = v_lo
                chi_v[rr, dr, pl.ds(16 * u, 16)] = v_hi
        return carry

    lax.fori_loop(0, CHUNK, build_const, 0)

    n_const = n_lo + n_hi

    def drain_one():
        pltpu.make_async_copy(
            out_hbm.at[pl.ds(0, CHUNK), pl.ds(ds0, 8), pl.ds(dj0, 128)],
            clo_v,
            sem_c,
        ).wait()

    def fire_const(c, carry):
        @pl.when(c < n_lo)
        def _():
            pltpu.async_copy(
                chi_v,
                out_hbm.at[pl.ds(CHUNK * c, CHUNK), pl.ds(ds0, 8), pl.ds(dj0, 128)],
                sem_c,
            )

        @pl.when(c >= n_lo)
        def _():
            pltpu.async_copy(
                clo_v,
                out_hbm.at[
                    pl.ds(tb_hi + CHUNK * (c - n_lo), CHUNK),
                    pl.ds(ds0, 8),
                    pl.ds(dj0, 128),
                ],
                sem_c,
            )

        @pl.when(c >= INFLIGHT)
        def _():
            drain_one()

        return carry

    lax.fori_loop(0, n_const, fire_const, 0)

    def band_chunk(c, carry):
        i0 = tb_lo + CHUNK * c

        def stage_row(rr, carry2):
            i = i0 + rr
            for dr in range(8):
                d = 8 * s + dr
                for u in range(8):
                    j = dj0 + 16 * u + lax.iota(jnp.int32, D)
                    m = jnp.clip(j - i, -MAX_REL, MAX_REL) + MAX_REL
                    band_v[rr, dr, pl.ds(16 * u, 16)] = plsc.load_gather(
                        table_v, [m * D + d]
                    )
            return carry2

        lax.fori_loop(0, CHUNK, stage_row, 0)
        pltpu.async_copy(
            band_v,
            out_hbm.at[pl.ds(i0, CHUNK), pl.ds(ds0, 8), pl.ds(dj0, 128)],
            sem_b,
        ).wait()
        return carry

    lax.fori_loop(0, n_band, band_chunk, 0)

    def drain(c, carry):
        drain_one()
        return carry

    lax.fori_loop(0, jnp.minimum(n_const, INFLIGHT), drain, 0)


def kernel(length, emb_table):
    del length
    out = _rpe_sc(emb_table.reshape(TABLE_ROWS * D))
    return jnp.transpose(out, (0, 2, 1))


# speedup vs baseline: 100.6397x; 1.0489x over previous
import functools

import jax
import jax.numpy as jnp
from jax import lax
from jax.experimental import pallas as pl
from jax.experimental.pallas import tpu as pltpu
from jax.experimental.pallas import tpu_sc as plsc

MAX_REL = 32
D = 16
L = 2048
TABLE_ROWS = 2 * MAX_REL + 1
CHUNK = 32
NUM_CORES = 2
NUM_SUBCORES = 16
INFLIGHT = 16

_mesh = plsc.VectorSubcoreMesh(
    core_axis_name="c",
    subcore_axis_name="s",
    num_cores=NUM_CORES,
    num_subcores=NUM_SUBCORES,
)


@functools.partial(
    pl.kernel,
    out_type=jax.ShapeDtypeStruct((L, D, L), jnp.float32),
    mesh=_mesh,
    compiler_params=pltpu.CompilerParams(needs_layout_passes=False),
    scratch_types=[
        pltpu.VMEM((TABLE_ROWS * D,), jnp.float32),
        pltpu.VMEM((CHUNK, 8, 128), jnp.float32),
        pltpu.VMEM((CHUNK, 8, 128), jnp.float32),
        pltpu.VMEM((CHUNK, 8, 128), jnp.float32),
        pltpu.SemaphoreType.DMA,
        pltpu.SemaphoreType.DMA,
    ],
)
def _rpe_sc(table_hbm, out_hbm, table_v, clo_v, chi_v, band_v, sem_c, sem_b):
    t = lax.axis_index("s")
    s = lax.axis_index("c")
    ds0 = pl.multiple_of(8 * s, 8)
    dj0 = pl.multiple_of(128 * t, 128)

    pltpu.sync_copy(table_hbm, table_v)

    tb_lo = jnp.maximum(128 * t - MAX_REL, 0)
    tb_hi = jnp.minimum(128 * t + 128 + MAX_REL, L)
    n_lo = tb_lo // CHUNK
    n_hi = (L - tb_hi) // CHUNK
    n_band = (tb_hi - tb_lo) // CHUNK

    def build_const(rr, carry):
        for dr in range(8):
            lane = jnp.full((D,), 8 * s + dr, jnp.int32)
            v_lo = plsc.load_gather(table_v, [lane])
            v_hi = plsc.load_gather(table_v, [(TABLE_ROWS - 1) * D + lane])
            for u in range(8):
                clo_v[rr, dr, pl.ds(16 * u, 16)] = v_lo
                chi_v[rr, dr, pl.ds(16 * u, 16)] = v_hi
        return carry

    lax.fori_loop(0, CHUNK, build_const, 0)

    n_const = n_lo + n_hi

    def drain_one():
        pltpu.make_async_copy(
            out_hbm.at[pl.ds(0, CHUNK), pl.ds(ds0, 8), pl.ds(dj0, 128)],
            clo_v,
            sem_c,
        ).wait()

    def fire_const(c, carry):
        @pl.when(c < n_lo)
        def _():
            pltpu.async_copy(
                chi_v,
                out_hbm.at[pl.ds(CHUNK * c, CHUNK), pl.ds(ds0, 8), pl.ds(dj0, 128)],
                sem_c,
            )

        @pl.when(c >= n_lo)
        def _():
            pltpu.async_copy(
                clo_v,
                out_hbm.at[
                    pl.ds(tb_hi + CHUNK * (c - n_lo), CHUNK),
                    pl.ds(ds0, 8),
                    pl.ds(dj0, 128),
                ],
                sem_c,
            )

        @pl.when(c >= INFLIGHT)
        def _():
            drain_one()

        return carry

    lax.fori_loop(0, n_const, fire_const, 0)

    def band_chunk(c, carry):
        i0 = tb_lo + CHUNK * c

        def stage_row(rr, carry2):
            i = i0 + rr
            for dr in range(8):
                d = 8 * s + dr
                for u in range(8):
                    j = dj0 + 16 * u + lax.iota(jnp.int32, D)
                    m = jnp.clip(j - i, -MAX_REL, MAX_REL) + MAX_REL
                    band_v[rr, dr, pl.ds(16 * u, 16)] = plsc.load_gather(
                        table_v, [m * D + d]
                    )
            return carry2

        lax.fori_loop(0, CHUNK, stage_row, 0)
        pltpu.async_copy(
            band_v,
            out_hbm.at[pl.ds(i0, CHUNK), pl.ds(ds0, 8), pl.ds(dj0, 128)],
            sem_b,
        ).wait()
        return carry

    lax.fori_loop(0, n_band, band_chunk, 0)

    def drain(c, carry):
        drain_one()
        return carry

    lax.fori_loop(0, jnp.minimum(n_const, INFLIGHT), drain, 0)


def kernel(length, emb_table):
    del length
    out = _rpe_sc(emb_table.reshape(TABLE_ROWS * D))
    return jnp.transpose(out, (0, 2, 1))
